# baseline (speedup 1.0000x reference)
import functools

import jax
import jax.numpy as jnp
from jax import lax
from jax.experimental import pallas as pl
from jax.experimental.pallas import tpu as pltpu
from jax.experimental.pallas import tpu_sc as plsc

H = 64
NC = 2
NS = 16
NW = NC * NS
CHUNK = 128
LANES = 16


def _body(x_hbm, table_hbm, out_hbm, idx_v, rows_v, sem):
    n_total = x_hbm.shape[0]
    n_w = n_total // NW
    n_chunks = n_w // CHUNK
    wid = lax.axis_index("s") * NC + lax.axis_index("c")
    base = wid * n_w

    def chunk_body(ci, _):
        off = base + ci * CHUNK
        pltpu.sync_copy(x_hbm.at[pl.ds(off, CHUNK)], idx_v)
        pltpu.async_copy(table_hbm.at[idx_v], rows_v, sem).wait()

        def group_body(g, _):
            idxs = idx_v[pl.ds(g * LANES, LANES)]
            m = jnp.where(idxs == 0, jnp.float32(0.0), jnp.float32(8.0))
            for j in range(LANES):
                mj = m[j]
                row = g * LANES + j
                for c in range(H // LANES):
                    sl = pl.ds(c * LANES, LANES)
                    rows_v[row, sl] = rows_v[row, sl] * mj
            return ()

        lax.fori_loop(0, CHUNK // LANES, group_body, ())
        pltpu.sync_copy(rows_v, out_hbm.at[pl.ds(off, CHUNK)])
        return ()

    lax.fori_loop(0, n_chunks, chunk_body, (), unroll=False)


def kernel(x, shared_weights):
    b, l = x.shape
    xf = x.reshape(b * l).astype(jnp.int32)
    call = functools.partial(
        pl.kernel,
        mesh=plsc.VectorSubcoreMesh(core_axis_name="c", subcore_axis_name="s"),
        out_type=jax.ShapeDtypeStruct((b * l, H), jnp.float32),
        scratch_types=[
            pltpu.VMEM((CHUNK,), jnp.int32),
            pltpu.VMEM((CHUNK, H), jnp.float32),
            pltpu.SemaphoreType.DMA,
        ],
        compiler_params=pltpu.CompilerParams(use_tc_tiling_on_sc=False),
    )(_body)
    out = call(xf, shared_weights)
    return out.reshape(b, l, H)


# speedup vs baseline: 1.5114x; 1.5114x over previous
import functools

import jax
import jax.numpy as jnp
from jax import lax
from jax.experimental import pallas as pl
from jax.experimental.pallas import tpu as pltpu
from jax.experimental.pallas import tpu_sc as plsc

H = 64
NC = 2
NS = 16
NW = NC * NS
CHUNK = 128
LANES = 16
NBUF = 4


def _body(x_hbm, table_hbm, out_hbm, idx_v, gbufs, obufs, gsems, osems):
    n_total = x_hbm.shape[0]
    n_w = n_total // NW
    n_chunks = n_w // CHUNK
    n_outer = n_chunks // NBUF
    wid = lax.axis_index("s") * NC + lax.axis_index("c")
    base = wid * n_w

    pltpu.sync_copy(x_hbm.at[pl.ds(base, n_w)], idx_v)

    def start_gather(ci, b):
        pltpu.async_copy(
            table_hbm.at[idx_v.at[pl.ds(ci * CHUNK, CHUNK)]],
            gbufs[b], gsems[b])

    for b in range(NBUF):
        start_gather(b, b)

    def compute_chunk(ci, b):
        gbuf, obuf = gbufs[b], obufs[b]

        def group_body(g, _):
            idxs = idx_v[pl.ds(ci * CHUNK + g * LANES, LANES)]
            m = jnp.where(idxs == 0, jnp.float32(0.0), jnp.float32(8.0))
            for j in range(LANES):
                mj = m[j]
                row = g * LANES + j
                for c in range(H // LANES):
                    sl = pl.ds(c * LANES, LANES)
                    obuf[row, sl] = gbuf[row, sl] * mj
            return ()

        lax.fori_loop(0, CHUNK // LANES, group_body, ())

    def wait_gather(b):
        pltpu.make_async_copy(
            table_hbm.at[pl.ds(0, CHUNK)], gbufs[b], gsems[b]).wait()

    def wait_out(b):
        pltpu.make_async_copy(
            obufs[b], out_hbm.at[pl.ds(0, CHUNK)], osems[b]).wait()

    def outer_body(k, _):
        for b in range(NBUF):
            ci = k * NBUF + b
            wait_gather(b)

            @pl.when(k > 0)
            def _():
                wait_out(b)

            compute_chunk(ci, b)
            pltpu.async_copy(
                obufs[b], out_hbm.at[pl.ds(base + ci * CHUNK, CHUNK)],
                osems[b])

            @pl.when(ci + NBUF < n_chunks)
            def _():
                start_gather(ci + NBUF, b)
        return ()

    lax.fori_loop(0, n_outer, outer_body, ())
    for b in range(NBUF):
        wait_out(b)


def kernel(x, shared_weights):
    b, l = x.shape
    xf = x.reshape(b * l).astype(jnp.int32)
    n_w = (b * l) // NW
    call = functools.partial(
        pl.kernel,
        mesh=plsc.VectorSubcoreMesh(core_axis_name="c", subcore_axis_name="s"),
        out_type=jax.ShapeDtypeStruct((b * l, H), jnp.float32),
        scratch_types=[
            pltpu.VMEM((n_w,), jnp.int32),
            [pltpu.VMEM((CHUNK, H), jnp.float32) for _ in range(NBUF)],
            [pltpu.VMEM((CHUNK, H), jnp.float32) for _ in range(NBUF)],
            [pltpu.SemaphoreType.DMA for _ in range(NBUF)],
            [pltpu.SemaphoreType.DMA for _ in range(NBUF)],
        ],
        compiler_params=pltpu.CompilerParams(use_tc_tiling_on_sc=False),
    )(_body)
    out = call(xf, shared_weights)
    return out.reshape(b, l, H)
